# Initial kernel scaffold; baseline (speedup 1.0000x reference)
import jax
import jax.numpy as jnp
from jax.experimental import pallas as pl


def kernel(x, lookup_table_0, lookup_table_1, lookup_table_2, proj_W_0, proj_W_1, proj_W_2):
    raise NotImplementedError("write your pallas kernel here")


# speedup vs baseline: 3.9435x; 3.9435x over previous
import functools

import jax
import jax.numpy as jnp
from jax import lax
from jax.experimental import pallas as pl
from jax.experimental.pallas import tpu as pltpu
from jax.experimental.pallas import tpu_sc as plsc

N_TOKEN = 100000
D_EMBED = 128
D_PROJ = 128
EMB_SCALE = float(D_PROJ) ** 0.5

R = 2000
NB0, NB1, NB2 = 20000 // R, 30000 // R, 50000 // R
NB = NB0 + NB1 + NB2


def _proj_body(t0, t1, t2, w0, w1, w2, out):
    pid = pl.program_id(0)

    @pl.when(pid < NB0)
    def _():
        out[...] = lax.dot(t0[...], w0[...],
                           preferred_element_type=jnp.float32) * EMB_SCALE

    @pl.when(jnp.logical_and(pid >= NB0, pid < NB0 + NB1))
    def _():
        out[...] = lax.dot(t1[...], w1[...],
                           preferred_element_type=jnp.float32) * EMB_SCALE

    @pl.when(pid >= NB0 + NB1)
    def _():
        out[...] = lax.dot(t2[...], w2[...],
                           preferred_element_type=jnp.float32) * EMB_SCALE


def _project(lt0, lt1, lt2, w0, w1, w2):
    return pl.pallas_call(
        _proj_body,
        grid=(NB,),
        in_specs=[
            pl.BlockSpec((R, 128), lambda i: (jnp.minimum(i, NB0 - 1), 0)),
            pl.BlockSpec((R, 32), lambda i: (jnp.clip(i - NB0, 0, NB1 - 1), 0)),
            pl.BlockSpec((R, 8), lambda i: (jnp.clip(i - NB0 - NB1, 0, NB2 - 1), 0)),
            pl.BlockSpec((128, 128), lambda i: (0, 0)),
            pl.BlockSpec((32, 128), lambda i: (0, 0)),
            pl.BlockSpec((8, 128), lambda i: (0, 0)),
        ],
        out_specs=pl.BlockSpec((R, 128), lambda i: (i, 0)),
        out_shape=jax.ShapeDtypeStruct((N_TOKEN, D_PROJ), jnp.float32),
    )(lt0, lt1, lt2, w0, w1, w2)


NC, NS = 2, 16
NW = NC * NS
CH = 128


def _make_gather(batch):
    assert batch % (NW * CH) == 0
    bpw = batch // NW
    nch = bpw // CH
    mesh = plsc.VectorSubcoreMesh(core_axis_name="c", subcore_axis_name="s")

    @functools.partial(
        pl.kernel,
        mesh=mesh,
        out_type=jax.ShapeDtypeStruct((batch, D_PROJ), jnp.float32),
        scratch_types=[
            pltpu.VMEM((nch, CH), jnp.int32),
            pltpu.VMEM((CH, D_PROJ), jnp.float32),
            pltpu.SemaphoreType.DMA,
        ],
    )
    def _gather(table_hbm, idx_hbm, out_hbm, idx_v, rows_v, sem):
        wid = lax.axis_index("s") * NC + lax.axis_index("c")
        base = wid * bpw
        pltpu.sync_copy(idx_hbm.at[wid], idx_v)

        def body(c, carry):
            pltpu.async_copy(table_hbm.at[idx_v.at[c]], rows_v, sem).wait()
            pltpu.sync_copy(rows_v, out_hbm.at[pl.ds(base + c * CH, CH)])
            return carry

        lax.fori_loop(0, nch, body, 0)

    return _gather


def kernel(x, lookup_table_0, lookup_table_1, lookup_table_2,
           proj_W_0, proj_W_1, proj_W_2):
    cat = _project(lookup_table_0, lookup_table_1, lookup_table_2,
                   proj_W_0, proj_W_1, proj_W_2)
    b, h = x.shape
    batch = b * h
    xr = x.reshape(NW, batch // (NW * CH), CH)
    y = _make_gather(batch)(cat, xr)
    return y.reshape(b, h, D_PROJ)


# speedup vs baseline: 4.7267x; 1.1986x over previous
import functools

import jax
import jax.numpy as jnp
from jax import lax
from jax.experimental import pallas as pl
from jax.experimental.pallas import tpu as pltpu
from jax.experimental.pallas import tpu_sc as plsc

N_TOKEN = 100000
D_EMBED = 128
D_PROJ = 128
EMB_SCALE = float(D_PROJ) ** 0.5

R = 2000
NB0, NB1, NB2 = 20000 // R, 30000 // R, 50000 // R
NB = NB0 + NB1 + NB2


def _proj_body(t0, t1, t2, w0, w1, w2, out):
    pid = pl.program_id(0)

    @pl.when(pid < NB0)
    def _():
        out[...] = lax.dot(t0[...], w0[...],
                           preferred_element_type=jnp.float32) * EMB_SCALE

    @pl.when(jnp.logical_and(pid >= NB0, pid < NB0 + NB1))
    def _():
        out[...] = lax.dot(t1[...], w1[...],
                           preferred_element_type=jnp.float32) * EMB_SCALE

    @pl.when(pid >= NB0 + NB1)
    def _():
        out[...] = lax.dot(t2[...], w2[...],
                           preferred_element_type=jnp.float32) * EMB_SCALE


def _project(lt0, lt1, lt2, w0, w1, w2):
    return pl.pallas_call(
        _proj_body,
        grid=(NB,),
        in_specs=[
            pl.BlockSpec((R, 128), lambda i: (jnp.minimum(i, NB0 - 1), 0)),
            pl.BlockSpec((R, 32), lambda i: (jnp.clip(i - NB0, 0, NB1 - 1), 0)),
            pl.BlockSpec((R, 8), lambda i: (jnp.clip(i - NB0 - NB1, 0, NB2 - 1), 0)),
            pl.BlockSpec((128, 128), lambda i: (0, 0)),
            pl.BlockSpec((32, 128), lambda i: (0, 0)),
            pl.BlockSpec((8, 128), lambda i: (0, 0)),
        ],
        out_specs=pl.BlockSpec((R, 128), lambda i: (i, 0)),
        out_shape=jax.ShapeDtypeStruct((N_TOKEN, D_PROJ), jnp.float32),
    )(lt0, lt1, lt2, w0, w1, w2)


NC, NS = 2, 16
NW = NC * NS
CH = 128


def _make_gather(batch):
    assert batch % (NW * CH) == 0
    bpw = batch // NW
    nch = bpw // CH
    mesh = plsc.VectorSubcoreMesh(core_axis_name="c", subcore_axis_name="s")

    assert nch % 2 == 0

    @functools.partial(
        pl.kernel,
        mesh=mesh,
        out_type=jax.ShapeDtypeStruct((batch, D_PROJ), jnp.float32),
        scratch_types=[
            pltpu.VMEM((nch, CH), jnp.int32),
            pltpu.VMEM((CH, D_PROJ), jnp.float32),
            pltpu.VMEM((CH, D_PROJ), jnp.float32),
            pltpu.SemaphoreType.DMA,
            pltpu.SemaphoreType.DMA,
        ],
    )
    def _gather(table_hbm, idx_hbm, out_hbm, idx_v, rows0, rows1, sem0, sem1):
        wid = lax.axis_index("s") * NC + lax.axis_index("c")
        base = wid * bpw
        pltpu.sync_copy(idx_hbm.at[wid], idx_v)
        pltpu.async_copy(table_hbm.at[idx_v.at[0]], rows0, sem0)

        def body(j, carry):
            c = 2 * j
            pltpu.async_copy(table_hbm.at[idx_v.at[c + 1]], rows1, sem1)
            pltpu.make_async_copy(table_hbm.at[idx_v.at[c]], rows0, sem0).wait()
            pltpu.sync_copy(rows0, out_hbm.at[pl.ds(base + c * CH, CH)])

            @pl.when(c + 2 < nch)
            def _():
                pltpu.async_copy(table_hbm.at[idx_v.at[c + 2]], rows0, sem0)

            pltpu.make_async_copy(
                table_hbm.at[idx_v.at[c + 1]], rows1, sem1).wait()
            pltpu.sync_copy(rows1, out_hbm.at[pl.ds(base + (c + 1) * CH, CH)])
            return carry

        lax.fori_loop(0, nch // 2, body, 0)

    return _gather


def kernel(x, lookup_table_0, lookup_table_1, lookup_table_2,
           proj_W_0, proj_W_1, proj_W_2):
    cat = _project(lookup_table_0, lookup_table_1, lookup_table_2,
                   proj_W_0, proj_W_1, proj_W_2)
    b, h = x.shape
    batch = b * h
    xr = x.reshape(NW, batch // (NW * CH), CH)
    y = _make_gather(batch)(cat, xr)
    return y.reshape(b, h, D_PROJ)


# speedup vs baseline: 4.7670x; 1.0085x over previous
import functools

import jax
import jax.numpy as jnp
from jax import lax
from jax.experimental import pallas as pl
from jax.experimental.pallas import tpu as pltpu
from jax.experimental.pallas import tpu_sc as plsc

N_TOKEN = 100000
D_EMBED = 128
D_PROJ = 128
EMB_SCALE = float(D_PROJ) ** 0.5

R = 2000
NB0, NB1, NB2 = 20000 // R, 30000 // R, 50000 // R
NB = NB0 + NB1 + NB2


def _proj_body(t0, t1, t2, w0, w1, w2, out):
    pid = pl.program_id(0)

    @pl.when(pid < NB0)
    def _():
        out[...] = lax.dot(t0[...], w0[...],
                           preferred_element_type=jnp.float32) * EMB_SCALE

    @pl.when(jnp.logical_and(pid >= NB0, pid < NB0 + NB1))
    def _():
        out[...] = lax.dot(t1[...], w1[...],
                           preferred_element_type=jnp.float32) * EMB_SCALE

    @pl.when(pid >= NB0 + NB1)
    def _():
        out[...] = lax.dot(t2[...], w2[...],
                           preferred_element_type=jnp.float32) * EMB_SCALE


def _project(lt0, lt1, lt2, w0, w1, w2):
    return pl.pallas_call(
        _proj_body,
        grid=(NB,),
        in_specs=[
            pl.BlockSpec((R, 128), lambda i: (jnp.minimum(i, NB0 - 1), 0)),
            pl.BlockSpec((R, 32), lambda i: (jnp.clip(i - NB0, 0, NB1 - 1), 0)),
            pl.BlockSpec((R, 8), lambda i: (jnp.clip(i - NB0 - NB1, 0, NB2 - 1), 0)),
            pl.BlockSpec((128, 128), lambda i: (0, 0)),
            pl.BlockSpec((32, 128), lambda i: (0, 0)),
            pl.BlockSpec((8, 128), lambda i: (0, 0)),
        ],
        out_specs=pl.BlockSpec((R, 128), lambda i: (i, 0)),
        out_shape=jax.ShapeDtypeStruct((N_TOKEN, D_PROJ), jnp.float32),
    )(lt0, lt1, lt2, w0, w1, w2)


NC, NS = 2, 16
NW = NC * NS
CH = 128


def _make_gather(batch):
    assert batch % (NW * CH) == 0
    bpw = batch // NW
    nch = bpw // CH
    mesh = plsc.VectorSubcoreMesh(core_axis_name="c", subcore_axis_name="s")

    NBUF = 4
    nmain = (nch - 2) // NBUF * NBUF
    assert nch - nmain >= 2

    @functools.partial(
        pl.kernel,
        mesh=mesh,
        out_type=jax.ShapeDtypeStruct((batch, D_PROJ), jnp.float32),
        scratch_types=[
            pltpu.VMEM((nch, CH), jnp.int32),
            pltpu.VMEM((NBUF, CH, D_PROJ), jnp.float32),
            pltpu.SemaphoreType.DMA,
            pltpu.SemaphoreType.DMA,
            pltpu.SemaphoreType.DMA,
            pltpu.SemaphoreType.DMA,
            pltpu.SemaphoreType.DMA,
            pltpu.SemaphoreType.DMA,
            pltpu.SemaphoreType.DMA,
            pltpu.SemaphoreType.DMA,
        ],
    )
    def _gather(table_hbm, idx_hbm, out_hbm, idx_v, rows,
                g0, g1, g2, g3, s0, s1, s2, s3):
        wid = lax.axis_index("s") * NC + lax.axis_index("c")
        base = wid * bpw
        gsem = [g0, g1, g2, g3]
        ssem = [s0, s1, s2, s3]
        pltpu.sync_copy(idx_hbm.at[wid], idx_v)

        def fire_g(c, b):
            pltpu.async_copy(table_hbm.at[idx_v.at[c]], rows.at[b], gsem[b])

        def wait_g(c, b):
            pltpu.make_async_copy(
                table_hbm.at[idx_v.at[c]], rows.at[b], gsem[b]).wait()

        def fire_s(c, b):
            pltpu.async_copy(
                rows.at[b], out_hbm.at[pl.ds(base + c * CH, CH)], ssem[b])

        def wait_s(c, b):
            pltpu.make_async_copy(
                rows.at[b], out_hbm.at[pl.ds(base + c * CH, CH)], ssem[b]).wait()

        fire_g(0, 0)
        fire_g(1, 1)

        def body(j, carry):
            k = NBUF * j
            for i in range(NBUF):
                wait_g(k + i, i)
                fire_s(k + i, i)
                nb = (i + 2) % NBUF

                @pl.when(k + i >= 2)
                def _():
                    wait_s(k + i - 2, nb)

                fire_g(k + i + 2, nb)
            return carry

        lax.fori_loop(0, nmain // NBUF, body, 0)
        for k in range(nmain, nch):
            b = k % NBUF
            wait_g(k, b)
            fire_s(k, b)
        for k in range(nch - NBUF, nch):
            wait_s(k, k % NBUF)

    return _gather


def kernel(x, lookup_table_0, lookup_table_1, lookup_table_2,
           proj_W_0, proj_W_1, proj_W_2):
    cat = _project(lookup_table_0, lookup_table_1, lookup_table_2,
                   proj_W_0, proj_W_1, proj_W_2)
    b, h = x.shape
    batch = b * h
    xr = x.reshape(NW, batch // (NW * CH), CH)
    y = _make_gather(batch)(cat, xr)
    return y.reshape(b, h, D_PROJ)
